# baseline (speedup 1.0000x reference)
import functools

import jax
import jax.numpy as jnp
from jax import lax
from jax.experimental import pallas as pl
from jax.experimental.pallas import tpu as pltpu
from jax.experimental.pallas import tpu_sc as plsc

HIDDEN = 1024
FF = 768
E = 64
TOPK = 2
T = 2048
R = 64
NCH = 128
PADROWS = NCH * R
NSLOT = TOPK * T
NEG = -1e30


def _router_dispatch_body(x_ref, gw_ref, pos_ref, w_ref, ce_ref, tot_ref):
    x = x_ref[...]
    gw = gw_ref[...]
    logits = lax.dot_general(x, gw, (((1,), (1,)), ((), ())),
                             preferred_element_type=jnp.float32)
    i1 = jnp.argmax(logits, axis=-1).astype(jnp.int32)
    m1 = jnp.max(logits, axis=-1)
    col = lax.broadcasted_iota(jnp.int32, logits.shape, 1)
    masked = jnp.where(col == i1[:, None], NEG, logits)
    i2 = jnp.argmax(masked, axis=-1).astype(jnp.int32)
    m2 = jnp.max(masked, axis=-1)
    w1 = 1.0 / (1.0 + jnp.exp(m2 - m1))

    NR = NSLOT // 128
    e2d = jnp.concatenate(
        [i1.reshape(NR // 2, 128), i2.reshape(NR // 2, 128)], axis=0)
    w_ref[...] = jnp.concatenate(
        [w1.reshape(NR // 2, 128), (1.0 - w1).reshape(NR // 2, 128)], axis=0)

    r128 = lax.broadcasted_iota(jnp.int32, (128, 128), 0)
    c128 = lax.broadcasted_iota(jnp.int32, (128, 128), 1)
    lt_strict = (r128 < c128).astype(jnp.float32)
    ones128 = jnp.ones((128, 128), jnp.float32)
    rr = lax.broadcasted_iota(jnp.int32, (NR, NR), 0)
    cc = lax.broadcasted_iota(jnp.int32, (NR, NR), 1)
    slt_rows = (rr > cc).astype(jnp.float32)
    dn = (((1,), (0,)), ((), ()))

    rank = jnp.zeros((NR, 128), jnp.float32)
    counts = jnp.zeros((1, 128), jnp.float32)
    lane64 = lax.broadcasted_iota(jnp.int32, (1, 128), 1)
    for e in range(E):
        mi = (e2d == e).astype(jnp.float32)
        lane_excl = lax.dot_general(mi, lt_strict, dn,
                                    preferred_element_type=jnp.float32)
        rt = lax.dot_general(mi, ones128, dn,
                             preferred_element_type=jnp.float32)
        row_excl = lax.dot_general(slt_rows, rt,
                                   (((1,), (0,)), ((), ())),
                                   preferred_element_type=jnp.float32)
        rank = rank + mi * (lane_excl + row_excl)
        counts = counts + jnp.where(lane64 == e, jnp.sum(mi), 0.0)

    nch = jnp.maximum(1.0, jnp.ceil(counts / R))
    nch = jnp.where(lane64 < E, nch, 0.0)
    le128 = (r128 <= c128).astype(jnp.float32)
    chunk_cum = lax.dot_general(nch, le128, dn,
                                preferred_element_type=jnp.float32)
    pad_base = (chunk_cum - nch) * R

    pb = jnp.zeros((NR, 128), jnp.float32)
    for e in range(E):
        pb_e = jnp.sum(jnp.where(lane64 == e, pad_base, 0.0))
        pb = pb + (e2d == e).astype(jnp.float32) * pb_e
    pos_ref[...] = (pb + rank).astype(jnp.int32)

    total = jnp.sum(nch)
    ce = jnp.zeros((1, 128), jnp.float32)
    for e in range(E):
        base_e = jnp.sum(jnp.where(lane64 == e, pad_base, 0.0)) / R
        n_e = jnp.sum(jnp.where(lane64 == e, nch, 0.0))
        in_rng = (lane64.astype(jnp.float32) >= base_e) & (
            lane64.astype(jnp.float32) < base_e + n_e)
        ce = ce + jnp.where(in_rng, float(e), 0.0)
    ce = jnp.where(lane64.astype(jnp.float32) < total, ce, float(E - 1))
    ce_ref[...] = ce.astype(jnp.int32)
    tot_ref[...] = total.astype(jnp.int32).reshape(1, 1)


def _router_dispatch(x, gate_w):
    return pl.pallas_call(
        _router_dispatch_body,
        out_shape=(
            jax.ShapeDtypeStruct((NSLOT // 128, 128), jnp.int32),
            jax.ShapeDtypeStruct((NSLOT // 128, 128), jnp.float32),
            jax.ShapeDtypeStruct((1, 128), jnp.int32),
            jax.ShapeDtypeStruct((1, 1), jnp.int32),
        ),
    )(x, gate_w)


def _sc_gather(idx, table):
    B = idx.shape[0]
    D = table.shape[1]
    info = plsc.get_sparse_core_info()
    nc, ns = info.num_cores, info.num_subcores
    nw = nc * ns
    b_per_w = B // nw
    gc = 48 if b_per_w % 48 == 0 else 32
    nit = b_per_w // gc
    mesh = plsc.VectorSubcoreMesh(core_axis_name="c", subcore_axis_name="s")

    @functools.partial(
        pl.kernel,
        out_type=jax.ShapeDtypeStruct((B, D), jnp.float32),
        mesh=mesh,
        scratch_types=[
            pltpu.VMEM((b_per_w,), jnp.int32),
            pltpu.VMEM((gc, D), jnp.float32),
            pltpu.VMEM((gc, D), jnp.float32),
            pltpu.SemaphoreType.DMA,
            pltpu.SemaphoreType.DMA,
            pltpu.SemaphoreType.DMA,
            pltpu.SemaphoreType.DMA,
        ],
    )
    def k(idx_hbm, table_hbm, out_hbm, idx_v, b0, b1, g0, g1, s0, s1):
        wid = lax.axis_index("s") * nc + lax.axis_index("c")
        base = wid * b_per_w
        bufs = (b0, b1)
        gsem = (g0, g1)
        ssem = (s0, s1)
        pltpu.sync_copy(idx_hbm.at[pl.ds(base, b_per_w)], idx_v)
        pltpu.async_copy(table_hbm.at[idx_v.at[pl.ds(0, gc)]], b0, g0)
        for c in range(nit):
            p = c % 2
            pltpu.make_async_copy(
                table_hbm.at[pl.ds(0, gc)], bufs[p], gsem[p]).wait()
            pltpu.async_copy(
                bufs[p], out_hbm.at[pl.ds(base + c * gc, gc)], ssem[p])
            if c + 1 < nit:
                if c >= 1:
                    pltpu.make_async_copy(
                        bufs[1 - p], out_hbm.at[pl.ds(0, gc)],
                        ssem[1 - p]).wait()
                pltpu.async_copy(
                    table_hbm.at[idx_v.at[pl.ds((c + 1) * gc, gc)]],
                    bufs[1 - p], gsem[1 - p])
        if nit > 1:
            pltpu.make_async_copy(
                bufs[(nit - 2) % 2], out_hbm.at[pl.ds(0, gc)],
                ssem[(nit - 2) % 2]).wait()
        pltpu.make_async_copy(
            bufs[(nit - 1) % 2], out_hbm.at[pl.ds(0, gc)],
            ssem[(nit - 1) % 2]).wait()

    return k(idx, table)


def _gmm_body(ce_ref, tot_ref, x_ref, w_ref, gp_ref, up_ref, dp_ref, out_ref):
    i = pl.program_id(0)

    @pl.when(i < tot_ref[0])
    def _():
        xs = x_ref[...]
        gp = gp_ref[0]
        up = up_ref[0]
        dp = dp_ref[0]
        dn = (((1,), (1,)), ((), ()))
        a = lax.dot_general(xs, gp, dn, preferred_element_type=jnp.float32)
        b = lax.dot_general(xs, up, dn, preferred_element_type=jnp.float32)
        h = (a * (1.0 / (1.0 + jnp.exp(-a)))) * b
        y = lax.dot_general(h, dp, dn, preferred_element_type=jnp.float32)
        out_ref[...] = y * w_ref[...]


def _gmm(chunk_expert, total, x_sorted, w_sorted, gate_proj, up_proj,
         down_proj):
    clamp = lambda i, tot: jnp.minimum(i, tot[0] - 1)
    grid_spec = pltpu.PrefetchScalarGridSpec(
        num_scalar_prefetch=2,
        grid=(NCH,),
        in_specs=[
            pl.BlockSpec((R, HIDDEN), lambda i, ce, tot: (clamp(i, tot), 0)),
            pl.BlockSpec((R, 1), lambda i, ce, tot: (clamp(i, tot), 0)),
            pl.BlockSpec((1, FF, HIDDEN), lambda i, ce, tot: (ce[i], 0, 0)),
            pl.BlockSpec((1, FF, HIDDEN), lambda i, ce, tot: (ce[i], 0, 0)),
            pl.BlockSpec((1, HIDDEN, FF), lambda i, ce, tot: (ce[i], 0, 0)),
        ],
        out_specs=pl.BlockSpec(
            (R, HIDDEN), lambda i, ce, tot: (clamp(i, tot), 0)),
    )
    return pl.pallas_call(
        _gmm_body,
        grid_spec=grid_spec,
        out_shape=jax.ShapeDtypeStruct((PADROWS, HIDDEN), jnp.float32),
    )(chunk_expert, total, x_sorted, w_sorted, gate_proj, up_proj, down_proj)


def _combine_body(ys_ref, out_ref):
    out_ref[...] = ys_ref[0] + ys_ref[1]


def _combine(ys):
    cb = 128
    return pl.pallas_call(
        _combine_body,
        grid=(T // cb,),
        in_specs=[pl.BlockSpec((2, cb, HIDDEN), lambda i: (0, i, 0))],
        out_specs=pl.BlockSpec((cb, HIDDEN), lambda i: (i, 0)),
        out_shape=jax.ShapeDtypeStruct((T, HIDDEN), jnp.float32),
    )(ys)


def kernel(hidden_states, gate_w, gate_proj, up_proj, down_proj):
    bsz, seq, hid = hidden_states.shape
    x = hidden_states.reshape(-1, hid)

    pos2d, w2d, ce_row, tot = _router_dispatch(x, gate_w)
    pos = pos2d.reshape(NSLOT)
    chunk_expert = ce_row.reshape(128)[:NCH]
    total = tot.reshape(1)

    tokf = (jnp.arange(NSLOT, dtype=jnp.int32) & (T - 1)).astype(jnp.float32)
    payload = jnp.stack([tokf, w2d.reshape(NSLOT)], axis=1)
    fill = ((jnp.arange(PADROWS, dtype=jnp.int32) * 7) & (T - 1)).astype(
        jnp.float32)
    img = jnp.stack([fill, jnp.zeros((PADROWS,), jnp.float32)], axis=1)
    img = img.at[pos].set(payload)
    row_token = img[:, 0].astype(jnp.int32)
    row_w = img[:, 1]

    x_sorted = _sc_gather(row_token, x)

    yw = _gmm(chunk_expert, total, x_sorted, row_w.reshape(PADROWS, 1),
              gate_proj, up_proj, down_proj)

    ys = _sc_gather(pos, yw).reshape(TOPK, T, HIDDEN)
    out = _combine(ys)
    return out.reshape(bsz, seq, hid)


# speedup vs baseline: 1.1613x; 1.1613x over previous
import functools

import jax
import jax.numpy as jnp
from jax import lax
from jax.experimental import pallas as pl
from jax.experimental.pallas import tpu as pltpu
from jax.experimental.pallas import tpu_sc as plsc

HIDDEN = 1024
FF = 768
E = 64
TOPK = 2
T = 2048
R = 128
NCH = 96
PADROWS = NCH * R
NSLOT = TOPK * T
NEG = -1e30


def _router_dispatch_body(x_ref, gw_ref, pos_ref, w_ref, ce_ref, tot_ref):
    x = x_ref[...]
    gw = gw_ref[...]
    logits = lax.dot_general(x, gw, (((1,), (1,)), ((), ())),
                             preferred_element_type=jnp.float32)
    i1 = jnp.argmax(logits, axis=-1).astype(jnp.int32)
    m1 = jnp.max(logits, axis=-1)
    col = lax.broadcasted_iota(jnp.int32, logits.shape, 1)
    masked = jnp.where(col == i1[:, None], NEG, logits)
    i2 = jnp.argmax(masked, axis=-1).astype(jnp.int32)
    m2 = jnp.max(masked, axis=-1)
    w1 = 1.0 / (1.0 + jnp.exp(m2 - m1))

    NR = NSLOT // 128
    e2d = jnp.concatenate(
        [i1.reshape(NR // 2, 128), i2.reshape(NR // 2, 128)], axis=0)
    w_ref[...] = jnp.concatenate(
        [w1.reshape(NR // 2, 128), (1.0 - w1).reshape(NR // 2, 128)], axis=0)

    r128 = lax.broadcasted_iota(jnp.int32, (128, 128), 0)
    c128 = lax.broadcasted_iota(jnp.int32, (128, 128), 1)
    lt_strict = (r128 < c128).astype(jnp.float32)
    ones128 = jnp.ones((128, 128), jnp.float32)
    rr = lax.broadcasted_iota(jnp.int32, (NR, NR), 0)
    cc = lax.broadcasted_iota(jnp.int32, (NR, NR), 1)
    slt_rows = (rr > cc).astype(jnp.float32)
    dn = (((1,), (0,)), ((), ()))

    rank = jnp.zeros((NR, 128), jnp.float32)
    counts = jnp.zeros((1, 128), jnp.float32)
    lane64 = lax.broadcasted_iota(jnp.int32, (1, 128), 1)
    for e in range(E):
        mi = (e2d == e).astype(jnp.float32)
        lane_excl = lax.dot_general(mi, lt_strict, dn,
                                    preferred_element_type=jnp.float32)
        rt = lax.dot_general(mi, ones128, dn,
                             preferred_element_type=jnp.float32)
        row_excl = lax.dot_general(slt_rows, rt,
                                   (((1,), (0,)), ((), ())),
                                   preferred_element_type=jnp.float32)
        rank = rank + mi * (lane_excl + row_excl)
        counts = counts + jnp.where(lane64 == e, jnp.sum(mi), 0.0)

    nch = jnp.maximum(1.0, jnp.ceil(counts / R))
    nch = jnp.where(lane64 < E, nch, 0.0)
    le128 = (r128 <= c128).astype(jnp.float32)
    chunk_cum = lax.dot_general(nch, le128, dn,
                                preferred_element_type=jnp.float32)
    pad_base = (chunk_cum - nch) * R

    pb = jnp.zeros((NR, 128), jnp.float32)
    for e in range(E):
        pb_e = jnp.sum(jnp.where(lane64 == e, pad_base, 0.0))
        pb = pb + (e2d == e).astype(jnp.float32) * pb_e
    pos_ref[...] = (pb + rank).astype(jnp.int32)

    total = jnp.sum(nch)
    ce = jnp.zeros((1, 128), jnp.float32)
    for e in range(E):
        base_e = jnp.sum(jnp.where(lane64 == e, pad_base, 0.0)) / R
        n_e = jnp.sum(jnp.where(lane64 == e, nch, 0.0))
        in_rng = (lane64.astype(jnp.float32) >= base_e) & (
            lane64.astype(jnp.float32) < base_e + n_e)
        ce = ce + jnp.where(in_rng, float(e), 0.0)
    ce = jnp.where(lane64.astype(jnp.float32) < total, ce, float(E - 1))
    ce_ref[...] = ce.astype(jnp.int32)
    tot_ref[...] = total.astype(jnp.int32).reshape(1, 1)


def _router_dispatch(x, gate_w):
    return pl.pallas_call(
        _router_dispatch_body,
        out_shape=(
            jax.ShapeDtypeStruct((NSLOT // 128, 128), jnp.int32),
            jax.ShapeDtypeStruct((NSLOT // 128, 128), jnp.float32),
            jax.ShapeDtypeStruct((1, 128), jnp.int32),
            jax.ShapeDtypeStruct((1, 1), jnp.int32),
        ),
    )(x, gate_w)


def _sc_gather(idx, table):
    B = idx.shape[0]
    D = table.shape[1]
    info = plsc.get_sparse_core_info()
    nc, ns = info.num_cores, info.num_subcores
    nw = nc * ns
    b_per_w = B // nw
    gc = 48 if b_per_w % 48 == 0 else 32
    nit = b_per_w // gc
    mesh = plsc.VectorSubcoreMesh(core_axis_name="c", subcore_axis_name="s")

    @functools.partial(
        pl.kernel,
        out_type=jax.ShapeDtypeStruct((B, D), jnp.float32),
        mesh=mesh,
        scratch_types=[
            pltpu.VMEM((b_per_w,), jnp.int32),
            pltpu.VMEM((gc, D), jnp.float32),
            pltpu.VMEM((gc, D), jnp.float32),
            pltpu.SemaphoreType.DMA,
            pltpu.SemaphoreType.DMA,
            pltpu.SemaphoreType.DMA,
            pltpu.SemaphoreType.DMA,
        ],
    )
    def k(idx_hbm, table_hbm, out_hbm, idx_v, b0, b1, g0, g1, s0, s1):
        wid = lax.axis_index("s") * nc + lax.axis_index("c")
        base = wid * b_per_w
        bufs = (b0, b1)
        gsem = (g0, g1)
        ssem = (s0, s1)
        pltpu.sync_copy(idx_hbm.at[pl.ds(base, b_per_w)], idx_v)
        pltpu.async_copy(table_hbm.at[idx_v.at[pl.ds(0, gc)]], b0, g0)
        for c in range(nit):
            p = c % 2
            pltpu.make_async_copy(
                table_hbm.at[pl.ds(0, gc)], bufs[p], gsem[p]).wait()
            pltpu.async_copy(
                bufs[p], out_hbm.at[pl.ds(base + c * gc, gc)], ssem[p])
            if c + 1 < nit:
                if c >= 1:
                    pltpu.make_async_copy(
                        bufs[1 - p], out_hbm.at[pl.ds(0, gc)],
                        ssem[1 - p]).wait()
                pltpu.async_copy(
                    table_hbm.at[idx_v.at[pl.ds((c + 1) * gc, gc)]],
                    bufs[1 - p], gsem[1 - p])
        if nit > 1:
            pltpu.make_async_copy(
                bufs[(nit - 2) % 2], out_hbm.at[pl.ds(0, gc)],
                ssem[(nit - 2) % 2]).wait()
        pltpu.make_async_copy(
            bufs[(nit - 1) % 2], out_hbm.at[pl.ds(0, gc)],
            ssem[(nit - 1) % 2]).wait()

    return k(idx, table)


def _gmm_body(ce_ref, tot_ref, x_ref, w_ref, gp_ref, up_ref, dp_ref, out_ref):
    i = pl.program_id(0)

    @pl.when(i < tot_ref[0])
    def _():
        xs = x_ref[...]
        gp = gp_ref[0]
        up = up_ref[0]
        dp = dp_ref[0]
        dn = (((1,), (1,)), ((), ()))
        a = lax.dot_general(xs, gp, dn, preferred_element_type=jnp.float32)
        b = lax.dot_general(xs, up, dn, preferred_element_type=jnp.float32)
        h = (a * (1.0 / (1.0 + jnp.exp(-a)))) * b
        y = lax.dot_general(h, dp, dn, preferred_element_type=jnp.float32)
        out_ref[...] = y * w_ref[...]


def _gmm(chunk_expert, total, x_sorted, w_sorted, gate_proj, up_proj,
         down_proj):
    clamp = lambda i, tot: jnp.minimum(i, tot[0] - 1)
    grid_spec = pltpu.PrefetchScalarGridSpec(
        num_scalar_prefetch=2,
        grid=(NCH,),
        in_specs=[
            pl.BlockSpec((R, HIDDEN), lambda i, ce, tot: (clamp(i, tot), 0)),
            pl.BlockSpec((R, 1), lambda i, ce, tot: (clamp(i, tot), 0)),
            pl.BlockSpec((1, FF, HIDDEN), lambda i, ce, tot: (ce[i], 0, 0)),
            pl.BlockSpec((1, FF, HIDDEN), lambda i, ce, tot: (ce[i], 0, 0)),
            pl.BlockSpec((1, HIDDEN, FF), lambda i, ce, tot: (ce[i], 0, 0)),
        ],
        out_specs=pl.BlockSpec(
            (R, HIDDEN), lambda i, ce, tot: (clamp(i, tot), 0)),
    )
    return pl.pallas_call(
        _gmm_body,
        grid_spec=grid_spec,
        out_shape=jax.ShapeDtypeStruct((PADROWS, HIDDEN), jnp.float32),
    )(chunk_expert, total, x_sorted, w_sorted, gate_proj, up_proj, down_proj)


def _combine_body(ys_ref, out_ref):
    out_ref[...] = ys_ref[0] + ys_ref[1]


def _combine(ys):
    cb = 128
    return pl.pallas_call(
        _combine_body,
        grid=(T // cb,),
        in_specs=[pl.BlockSpec((2, cb, HIDDEN), lambda i: (0, i, 0))],
        out_specs=pl.BlockSpec((cb, HIDDEN), lambda i: (i, 0)),
        out_shape=jax.ShapeDtypeStruct((T, HIDDEN), jnp.float32),
    )(ys)


def kernel(hidden_states, gate_w, gate_proj, up_proj, down_proj):
    bsz, seq, hid = hidden_states.shape
    x = hidden_states.reshape(-1, hid)

    pos2d, w2d, ce_row, tot = _router_dispatch(x, gate_w)
    pos = pos2d.reshape(NSLOT)
    chunk_expert = ce_row.reshape(128)[:NCH]
    total = tot.reshape(1)

    tokf = (jnp.arange(NSLOT, dtype=jnp.int32) & (T - 1)).astype(jnp.float32)
    payload = jnp.stack([tokf, w2d.reshape(NSLOT)], axis=1)
    fill = ((jnp.arange(PADROWS, dtype=jnp.int32) * 7) & (T - 1)).astype(
        jnp.float32)
    img = jnp.stack([fill, jnp.zeros((PADROWS,), jnp.float32)], axis=1)
    img = img.at[pos].set(payload)
    row_token = img[:, 0].astype(jnp.int32)
    row_w = img[:, 1]

    x_sorted = _sc_gather(row_token, x)

    yw = _gmm(chunk_expert, total, x_sorted, row_w.reshape(PADROWS, 1),
              gate_proj, up_proj, down_proj)

    ys = _sc_gather(pos, yw).reshape(TOPK, T, HIDDEN)
    out = _combine(ys)
    return out.reshape(bsz, seq, hid)


# speedup vs baseline: 1.1646x; 1.0028x over previous
import functools

import jax
import jax.numpy as jnp
from jax import lax
from jax.experimental import pallas as pl
from jax.experimental.pallas import tpu as pltpu
from jax.experimental.pallas import tpu_sc as plsc

HIDDEN = 1024
FF = 768
E = 64
TOPK = 2
T = 2048
R = 128
NCH = 96
PADROWS = NCH * R
NSLOT = TOPK * T
NEG = -1e30


def _router_dispatch_body(x_ref, gw_ref, pos_ref, w_ref, ce_ref, tot_ref):
    x = x_ref[...]
    gw = gw_ref[...]
    logits = lax.dot_general(x, gw, (((1,), (1,)), ((), ())),
                             preferred_element_type=jnp.float32)
    i1 = jnp.argmax(logits, axis=-1).astype(jnp.int32)
    m1 = jnp.max(logits, axis=-1)
    col = lax.broadcasted_iota(jnp.int32, logits.shape, 1)
    masked = jnp.where(col == i1[:, None], NEG, logits)
    i2 = jnp.argmax(masked, axis=-1).astype(jnp.int32)
    m2 = jnp.max(masked, axis=-1)
    w1 = 1.0 / (1.0 + jnp.exp(m2 - m1))

    NR = NSLOT // 128
    e2d = jnp.concatenate(
        [i1.reshape(NR // 2, 128), i2.reshape(NR // 2, 128)], axis=0)
    w_ref[...] = jnp.concatenate(
        [w1.reshape(NR // 2, 128), (1.0 - w1).reshape(NR // 2, 128)], axis=0)

    r128 = lax.broadcasted_iota(jnp.int32, (128, 128), 0)
    c128 = lax.broadcasted_iota(jnp.int32, (128, 128), 1)
    lt_strict = (r128 < c128).astype(jnp.float32)
    ones128 = jnp.ones((128, 128), jnp.float32)
    rr = lax.broadcasted_iota(jnp.int32, (NR, NR), 0)
    cc = lax.broadcasted_iota(jnp.int32, (NR, NR), 1)
    slt_rows = (rr > cc).astype(jnp.float32)
    dn = (((1,), (0,)), ((), ()))

    rank = jnp.zeros((NR, 128), jnp.float32)
    counts = jnp.zeros((1, 128), jnp.float32)
    lane64 = lax.broadcasted_iota(jnp.int32, (1, 128), 1)
    for e in range(E):
        mi = (e2d == e).astype(jnp.float32)
        lane_excl = lax.dot_general(mi, lt_strict, dn,
                                    preferred_element_type=jnp.float32)
        rt = lax.dot_general(mi, ones128, dn,
                             preferred_element_type=jnp.float32)
        row_excl = lax.dot_general(slt_rows, rt,
                                   (((1,), (0,)), ((), ())),
                                   preferred_element_type=jnp.float32)
        rank = rank + mi * (lane_excl + row_excl)
        counts = counts + jnp.where(lane64 == e, jnp.sum(mi), 0.0)

    nch = jnp.maximum(1.0, jnp.ceil(counts / R))
    nch = jnp.where(lane64 < E, nch, 0.0)
    le128 = (r128 <= c128).astype(jnp.float32)
    chunk_cum = lax.dot_general(nch, le128, dn,
                                preferred_element_type=jnp.float32)
    pad_base = (chunk_cum - nch) * R

    pb = jnp.zeros((NR, 128), jnp.float32)
    for e in range(E):
        pb_e = jnp.sum(jnp.where(lane64 == e, pad_base, 0.0))
        pb = pb + (e2d == e).astype(jnp.float32) * pb_e
    pos_ref[...] = (pb + rank).astype(jnp.int32)

    total = jnp.sum(nch)
    ce = jnp.zeros((1, 128), jnp.float32)
    for e in range(E):
        base_e = jnp.sum(jnp.where(lane64 == e, pad_base, 0.0)) / R
        n_e = jnp.sum(jnp.where(lane64 == e, nch, 0.0))
        in_rng = (lane64.astype(jnp.float32) >= base_e) & (
            lane64.astype(jnp.float32) < base_e + n_e)
        ce = ce + jnp.where(in_rng, float(e), 0.0)
    ce = jnp.where(lane64.astype(jnp.float32) < total, ce, float(E - 1))
    ce_ref[...] = ce.astype(jnp.int32)
    tot_ref[...] = total.astype(jnp.int32).reshape(1, 1)


def _router_dispatch(x, gate_w):
    return pl.pallas_call(
        _router_dispatch_body,
        out_shape=(
            jax.ShapeDtypeStruct((NSLOT // 128, 128), jnp.int32),
            jax.ShapeDtypeStruct((NSLOT // 128, 128), jnp.float32),
            jax.ShapeDtypeStruct((1, 128), jnp.int32),
            jax.ShapeDtypeStruct((1, 1), jnp.int32),
        ),
    )(x, gate_w)


def _sc_gather(idx, table):
    B = idx.shape[0]
    D = table.shape[1]
    info = plsc.get_sparse_core_info()
    nc, ns = info.num_cores, info.num_subcores
    nw = nc * ns
    b_per_w = B // nw
    gc = 48 if b_per_w % 48 == 0 else 32
    nit = b_per_w // gc
    mesh = plsc.VectorSubcoreMesh(core_axis_name="c", subcore_axis_name="s")

    @functools.partial(
        pl.kernel,
        out_type=jax.ShapeDtypeStruct((B, D), jnp.float32),
        mesh=mesh,
        scratch_types=[
            pltpu.VMEM((b_per_w,), jnp.int32),
            pltpu.VMEM((gc, D), jnp.float32),
            pltpu.VMEM((gc, D), jnp.float32),
            pltpu.SemaphoreType.DMA,
            pltpu.SemaphoreType.DMA,
            pltpu.SemaphoreType.DMA,
            pltpu.SemaphoreType.DMA,
        ],
    )
    def k(idx_hbm, table_hbm, out_hbm, idx_v, b0, b1, g0, g1, s0, s1):
        wid = lax.axis_index("s") * nc + lax.axis_index("c")
        base = wid * b_per_w
        bufs = (b0, b1)
        gsem = (g0, g1)
        ssem = (s0, s1)
        pltpu.sync_copy(idx_hbm.at[pl.ds(base, b_per_w)], idx_v)
        pltpu.async_copy(table_hbm.at[idx_v.at[pl.ds(0, gc)]], b0, g0)
        for c in range(nit):
            p = c % 2
            pltpu.make_async_copy(
                table_hbm.at[pl.ds(0, gc)], bufs[p], gsem[p]).wait()
            pltpu.async_copy(
                bufs[p], out_hbm.at[pl.ds(base + c * gc, gc)], ssem[p])
            if c + 1 < nit:
                if c >= 1:
                    pltpu.make_async_copy(
                        bufs[1 - p], out_hbm.at[pl.ds(0, gc)],
                        ssem[1 - p]).wait()
                pltpu.async_copy(
                    table_hbm.at[idx_v.at[pl.ds((c + 1) * gc, gc)]],
                    bufs[1 - p], gsem[1 - p])
        if nit > 1:
            pltpu.make_async_copy(
                bufs[(nit - 2) % 2], out_hbm.at[pl.ds(0, gc)],
                ssem[(nit - 2) % 2]).wait()
        pltpu.make_async_copy(
            bufs[(nit - 1) % 2], out_hbm.at[pl.ds(0, gc)],
            ssem[(nit - 1) % 2]).wait()

    return k(idx, table)


def _sc_gather_combine(pos, yw):
    info = plsc.get_sparse_core_info()
    nc, ns = info.num_cores, info.num_subcores
    nw = nc * ns
    tpw = T // nw
    gt = 32
    nh = tpw // gt
    mesh = plsc.VectorSubcoreMesh(core_axis_name="c", subcore_axis_name="s")

    @functools.partial(
        pl.kernel,
        out_type=jax.ShapeDtypeStruct((T, HIDDEN), jnp.float32),
        mesh=mesh,
        scratch_types=[
            pltpu.VMEM((tpw,), jnp.int32),
            pltpu.VMEM((tpw,), jnp.int32),
            pltpu.VMEM((gt, HIDDEN), jnp.float32),
            pltpu.VMEM((gt, HIDDEN), jnp.float32),
            pltpu.SemaphoreType.DMA,
            pltpu.SemaphoreType.DMA,
            pltpu.SemaphoreType.DMA,
        ],
    )
    def k(pos_hbm, yw_hbm, out_hbm, ia_v, ib_v, ba, bb, sa, sb, so):
        wid = lax.axis_index("s") * nc + lax.axis_index("c")
        tbase = wid * tpw
        pltpu.sync_copy(pos_hbm.at[pl.ds(tbase, tpw)], ia_v)
        pltpu.sync_copy(pos_hbm.at[pl.ds(T + tbase, tpw)], ib_v)
        for h in range(nh):
            if h >= 1:
                pltpu.make_async_copy(
                    ba, out_hbm.at[pl.ds(0, gt)], so).wait()
            pltpu.async_copy(yw_hbm.at[ia_v.at[pl.ds(h * gt, gt)]], ba, sa)
            pltpu.async_copy(yw_hbm.at[ib_v.at[pl.ds(h * gt, gt)]], bb, sb)
            pltpu.make_async_copy(yw_hbm.at[pl.ds(0, gt)], ba, sa).wait()
            pltpu.make_async_copy(yw_hbm.at[pl.ds(0, gt)], bb, sb).wait()

            def add_row(r, _):
                def add_chunk(c, _2):
                    sl = pl.ds(c * 16, 16)
                    ba[r, sl] = ba[r, sl] + bb[r, sl]
                    return 0
                lax.fori_loop(0, HIDDEN // 16, add_chunk, 0)
                return 0
            lax.fori_loop(0, gt, add_row, 0)
            pltpu.async_copy(ba, out_hbm.at[pl.ds(tbase + h * gt, gt)], so)
        pltpu.make_async_copy(ba, out_hbm.at[pl.ds(0, gt)], so).wait()

    return k(pos, yw)


def _gmm_body(ce_ref, tot_ref, x_ref, w_ref, gp_ref, up_ref, dp_ref, out_ref):
    i = pl.program_id(0)

    @pl.when(i < tot_ref[0])
    def _():
        xs = x_ref[...]
        gp = gp_ref[0]
        up = up_ref[0]
        dp = dp_ref[0]
        dn = (((1,), (1,)), ((), ()))
        a = lax.dot_general(xs, gp, dn, preferred_element_type=jnp.float32)
        b = lax.dot_general(xs, up, dn, preferred_element_type=jnp.float32)
        h = (a * (1.0 / (1.0 + jnp.exp(-a)))) * b
        y = lax.dot_general(h, dp, dn, preferred_element_type=jnp.float32)
        out_ref[...] = y * w_ref[...]


def _gmm(chunk_expert, total, x_sorted, w_sorted, gate_proj, up_proj,
         down_proj):
    clamp = lambda i, tot: jnp.minimum(i, tot[0] - 1)
    grid_spec = pltpu.PrefetchScalarGridSpec(
        num_scalar_prefetch=2,
        grid=(NCH,),
        in_specs=[
            pl.BlockSpec((R, HIDDEN), lambda i, ce, tot: (clamp(i, tot), 0)),
            pl.BlockSpec((R, 1), lambda i, ce, tot: (clamp(i, tot), 0)),
            pl.BlockSpec((1, FF, HIDDEN), lambda i, ce, tot: (ce[i], 0, 0)),
            pl.BlockSpec((1, FF, HIDDEN), lambda i, ce, tot: (ce[i], 0, 0)),
            pl.BlockSpec((1, HIDDEN, FF), lambda i, ce, tot: (ce[i], 0, 0)),
        ],
        out_specs=pl.BlockSpec(
            (R, HIDDEN), lambda i, ce, tot: (clamp(i, tot), 0)),
    )
    return pl.pallas_call(
        _gmm_body,
        grid_spec=grid_spec,
        out_shape=jax.ShapeDtypeStruct((PADROWS, HIDDEN), jnp.float32),
    )(chunk_expert, total, x_sorted, w_sorted, gate_proj, up_proj, down_proj)


def _combine_body(ys_ref, out_ref):
    out_ref[...] = ys_ref[0] + ys_ref[1]


def _combine(ys):
    cb = 128
    return pl.pallas_call(
        _combine_body,
        grid=(T // cb,),
        in_specs=[pl.BlockSpec((2, cb, HIDDEN), lambda i: (0, i, 0))],
        out_specs=pl.BlockSpec((cb, HIDDEN), lambda i: (i, 0)),
        out_shape=jax.ShapeDtypeStruct((T, HIDDEN), jnp.float32),
    )(ys)


def kernel(hidden_states, gate_w, gate_proj, up_proj, down_proj):
    bsz, seq, hid = hidden_states.shape
    x = hidden_states.reshape(-1, hid)

    pos2d, w2d, ce_row, tot = _router_dispatch(x, gate_w)
    pos = pos2d.reshape(NSLOT)
    chunk_expert = ce_row.reshape(128)[:NCH]
    total = tot.reshape(1)

    tokf = (jnp.arange(NSLOT, dtype=jnp.int32) & (T - 1)).astype(jnp.float32)
    payload = jnp.stack([tokf, w2d.reshape(NSLOT)], axis=1)
    fill = ((jnp.arange(PADROWS, dtype=jnp.int32) * 7) & (T - 1)).astype(
        jnp.float32)
    img = jnp.stack([fill, jnp.zeros((PADROWS,), jnp.float32)], axis=1)
    img = img.at[pos].set(payload, unique_indices=True,
                          mode="promise_in_bounds")
    row_token = img[:, 0].astype(jnp.int32)
    row_w = img[:, 1]

    x_sorted = _sc_gather(row_token, x)

    yw = _gmm(chunk_expert, total, x_sorted, row_w.reshape(PADROWS, 1),
              gate_proj, up_proj, down_proj)

    out = _sc_gather_combine(pos, yw)
    return out.reshape(bsz, seq, hid)
